# baseline (speedup 1.0000x reference)
import functools

import jax
import jax.numpy as jnp
from jax import lax
from jax.experimental import pallas as pl
from jax.experimental.pallas import tpu as pltpu
from jax.experimental.pallas import tpu_sc as plsc

NP = 50000
NA = 10000
DI = 128
H = 32

NC, NS = 2, 16
BLK = 128

PP_B = 196
AP_B = 52
EPP_PAD = PP_B * BLK * NS
EAP_PAD = AP_B * BLK * NS

NP_H = 50176
NA_H = 10240
PT = 3136
AT = 640

_mesh = plsc.VectorSubcoreMesh(core_axis_name="c", subcore_axis_name="s")
_sc_params = pltpu.CompilerParams(use_tc_tiling_on_sc=False)


def _zero_fill_2d(buf, rows):
    z = jnp.zeros((16,), jnp.float32)

    def st(i, _):
        buf[i, pl.ds(0, 16)] = z
        buf[i, pl.ds(16, 16)] = z
        return 0

    lax.fori_loop(0, rows, st, 0)


def _edge_loop(eidx, tab, acc, b0, nb, ib, rows0, rows1, si, sg0, sg1,
               ss0, ss1):
    last = b0 + nb - 1

    def iw(k):
        return pltpu.make_async_copy(eidx.at[b0], ib[k], si[k]).wait()

    def gather(k, rows, sg):
        pltpu.async_copy(tab.at[ib[k].at[0]], rows, sg)

    def gw(k, rows, sg):
        pltpu.make_async_copy(tab.at[ib[k].at[0]], rows, sg).wait()

    def scat(k, rows, ss):
        pltpu.async_copy(rows, acc.at[ib[k].at[1]], ss, add=True)

    def sw(k, rows, ss):
        pltpu.make_async_copy(rows, acc.at[ib[k].at[1]], ss).wait()

    for k in range(4):
        pltpu.async_copy(eidx.at[b0 + k], ib[k], si[k])
    iw(0)
    gather(0, rows0, sg0)
    iw(1)
    gather(1, rows1, sg1)

    def quad(i, _):
        b = b0 + 4 * i
        gw(0, rows0, sg0)
        scat(0, rows0, ss0)
        gw(1, rows1, sg1)
        scat(1, rows1, ss1)
        sw(0, rows0, ss0)
        iw(2)
        gather(2, rows0, sg0)
        pltpu.async_copy(eidx.at[jnp.minimum(b + 4, last)], ib[0], si[0])
        gw(2, rows0, sg0)
        scat(2, rows0, ss0)
        sw(1, rows1, ss1)
        iw(3)
        gather(3, rows1, sg1)
        pltpu.async_copy(eidx.at[jnp.minimum(b + 5, last)], ib[1], si[1])
        gw(3, rows1, sg1)
        scat(3, rows1, ss1)
        sw(2, rows0, ss0)
        iw(0)
        gather(0, rows0, sg0)
        pltpu.async_copy(eidx.at[jnp.minimum(b + 6, last)], ib[2], si[2])
        sw(3, rows1, ss1)
        iw(1)
        gather(1, rows1, sg1)
        pltpu.async_copy(eidx.at[jnp.minimum(b + 7, last)], ib[3], si[3])
        return 0

    lax.fori_loop(0, nb // 4, quad, 0)
    gw(0, rows0, sg0)
    gw(1, rows1, sg1)
    iw(2)
    iw(3)


def _hist_loop(eidx, cnt, b0, nb, ib, ones_v, si, ss0, ss1):
    last = b0 + nb - 1

    def iw(k):
        return pltpu.make_async_copy(eidx.at[b0], ib[k], si[k]).wait()

    def scat(k, ss):
        pltpu.async_copy(ones_v, cnt.at[ib[k].at[1]], ss, add=True)

    def sw(k, ss):
        pltpu.make_async_copy(ones_v, cnt.at[ib[k].at[1]], ss).wait()

    for k in range(4):
        pltpu.async_copy(eidx.at[b0 + k], ib[k], si[k])

    def quad(i, _):
        b = b0 + 4 * i
        iw(0)
        scat(0, ss0)
        iw(1)
        scat(1, ss1)
        sw(0, ss0)
        pltpu.async_copy(eidx.at[jnp.minimum(b + 4, last)], ib[0], si[0])
        iw(2)
        scat(2, ss0)
        sw(1, ss1)
        pltpu.async_copy(eidx.at[jnp.minimum(b + 5, last)], ib[1], si[1])
        iw(3)
        scat(3, ss1)
        sw(2, ss0)
        pltpu.async_copy(eidx.at[jnp.minimum(b + 6, last)], ib[2], si[2])
        sw(3, ss1)
        pltpu.async_copy(eidx.at[jnp.minimum(b + 7, last)], ib[3], si[3])
        return 0

    lax.fori_loop(0, nb // 4, quad, 0)
    for k in range(4):
        iw(k)


@functools.partial(
    pl.kernel,
    out_type=(
        jax.ShapeDtypeStruct((NP_H,), jnp.float32),
        jax.ShapeDtypeStruct((NP_H,), jnp.float32),
        jax.ShapeDtypeStruct((NA_H,), jnp.float32),
    ),
    mesh=_mesh,
    compiler_params=_sc_params,
    scratch_types=[
        pltpu.VMEM_SHARED((NP_H,), jnp.float32),
        pltpu.VMEM_SHARED((NA_H,), jnp.float32),
        pltpu.VMEM((2, BLK), jnp.int32),
        pltpu.VMEM((2, BLK), jnp.int32),
        pltpu.VMEM((2, BLK), jnp.int32),
        pltpu.VMEM((2, BLK), jnp.int32),
        pltpu.VMEM((BLK,), jnp.float32),
        pltpu.VMEM((PT,), jnp.float32),
        pltpu.SemaphoreType.DMA,
        pltpu.SemaphoreType.DMA,
        pltpu.SemaphoreType.DMA,
        pltpu.SemaphoreType.DMA,
        pltpu.SemaphoreType.DMA,
        pltpu.SemaphoreType.DMA,
    ],
)
def _sc_hist(e_pp, e_ap, e_pa, out_pp, out_ap, out_pa,
             cntA, cntB, ib0, ib1, ib2, ib3, ones_v, zbuf,
             si0, si1, si2, si3, ss0, ss1):
    ib = [ib0, ib1, ib2, ib3]
    si = [si0, si1, si2, si3]
    c = lax.axis_index("c")
    s = lax.axis_index("s")
    one = jnp.full((16,), 1.0, jnp.float32)
    z = jnp.zeros((16,), jnp.float32)
    for i in range(BLK // 16):
        ones_v[pl.ds(i * 16, 16)] = one

    def zf(i, _):
        zbuf[pl.ds(i * 16, 16)] = z
        return 0

    lax.fori_loop(0, PT // 16, zf, 0)
    pltpu.sync_copy(zbuf, cntA.at[pl.ds(s * PT, PT)])
    pltpu.sync_copy(zbuf.at[pl.ds(0, AT)], cntB.at[pl.ds(s * AT, AT)])
    plsc.subcore_barrier()

    @pl.when(c == 0)
    def _():
        _hist_loop(e_pp, cntA, s * PP_B, PP_B, ib, ones_v, si, ss0, ss1)

    @pl.when(c == 1)
    def _():
        _hist_loop(e_ap, cntA, s * AP_B, AP_B, ib, ones_v, si, ss0, ss1)
        _hist_loop(e_pa, cntB, s * AP_B, AP_B, ib, ones_v, si, ss0, ss1)

    plsc.subcore_barrier()

    @pl.when(c == 0)
    def _():
        pltpu.sync_copy(cntA.at[pl.ds(s * PT, PT)], zbuf)
        pltpu.sync_copy(zbuf, out_pp.at[pl.ds(s * PT, PT)])

    @pl.when(c == 1)
    def _():
        pltpu.sync_copy(cntA.at[pl.ds(s * PT, PT)], zbuf)
        pltpu.sync_copy(zbuf, out_ap.at[pl.ds(s * PT, PT)])
        pltpu.sync_copy(cntB.at[pl.ds(s * AT, AT)], zbuf.at[pl.ds(0, AT)])
        pltpu.sync_copy(zbuf.at[pl.ds(0, AT)], out_pa.at[pl.ds(s * AT, AT)])


@functools.partial(
    pl.kernel,
    out_type=(
        jax.ShapeDtypeStruct((NP_H, 4 * H), jnp.float32),
        jax.ShapeDtypeStruct((NA_H, 4 * H), jnp.float32),
    ),
    mesh=_mesh,
    compiler_params=_sc_params,
    scratch_types=[
        pltpu.VMEM_SHARED((NP_H, H), jnp.float32),
        pltpu.VMEM_SHARED((NA_H, H), jnp.float32),
        pltpu.VMEM((2, BLK), jnp.int32),
        pltpu.VMEM((2, BLK), jnp.int32),
        pltpu.VMEM((2, BLK), jnp.int32),
        pltpu.VMEM((2, BLK), jnp.int32),
        pltpu.VMEM((BLK, H), jnp.float32),
        pltpu.VMEM((BLK, H), jnp.float32),
        pltpu.SemaphoreType.DMA,
        pltpu.SemaphoreType.DMA,
        pltpu.SemaphoreType.DMA,
        pltpu.SemaphoreType.DMA,
        pltpu.SemaphoreType.DMA,
        pltpu.SemaphoreType.DMA,
        pltpu.SemaphoreType.DMA,
        pltpu.SemaphoreType.DMA,
    ],
)
def _sc_scatter(tab_p, tab_a, e_pp, e_ap, e_pa,
                out_p, out_a,
                accA, accB, ib0, ib1, ib2, ib3, rows0, rows1,
                si0, si1, si2, si3, sg0, sg1, ss0, ss1):
    ib = [ib0, ib1, ib2, ib3]
    si = [si0, si1, si2, si3]
    c = lax.axis_index("c")
    s = lax.axis_index("s")
    _zero_fill_2d(rows0, BLK)
    for k in range(PT // BLK):
        pltpu.sync_copy(rows0, accA.at[pl.ds(s * PT + k * BLK, BLK)])
    pltpu.sync_copy(rows0.at[pl.ds(0, PT % BLK)],
                    accA.at[pl.ds(s * PT + (PT // BLK) * BLK, PT % BLK)])
    for k in range(AT // BLK):
        pltpu.sync_copy(rows0, accB.at[pl.ds(s * AT + k * BLK, BLK)])
    plsc.subcore_barrier()

    @pl.when(c == 0)
    def _():
        _edge_loop(e_pp, tab_p, accA, s * PP_B, PP_B,
                   ib, rows0, rows1, si, sg0, sg1, ss0, ss1)

    @pl.when(c == 1)
    def _():
        _edge_loop(e_ap, tab_a, accA, s * AP_B, AP_B,
                   ib, rows0, rows1, si, sg0, sg1, ss0, ss1)
        _edge_loop(e_pa, tab_p, accB, s * AP_B, AP_B,
                   ib, rows0, rows1, si, sg0, sg1, ss0, ss1)

    plsc.subcore_barrier()

    def _drain(acc, out, col, base, n):
        for k in range(n // BLK):
            pltpu.sync_copy(acc.at[pl.ds(base + k * BLK, BLK)], rows0)
            pltpu.sync_copy(rows0, out.at[pl.ds(base + k * BLK, BLK),
                                          pl.ds(col, H)])
        rem = n % BLK
        if rem:
            off = base + (n // BLK) * BLK
            pltpu.sync_copy(acc.at[pl.ds(off, rem)], rows0.at[pl.ds(0, rem)])
            pltpu.sync_copy(rows0.at[pl.ds(0, rem)],
                            out.at[pl.ds(off, rem), pl.ds(col, H)])

    @pl.when(c == 0)
    def _():
        _drain(accA, out_p, 0, s * PT, PT)

    @pl.when(c == 1)
    def _():
        _drain(accA, out_p, H, s * PT, PT)
        _drain(accB, out_a, 0, s * AT, AT)


_RP = 5000
_RA = 5000


def _full(shape):
    return pl.BlockSpec(shape, lambda i: (0, 0))


def _tc_proj1p_body(x, deg, w, o):
    xw = jnp.dot(x[...], w[...], preferred_element_type=jnp.float32)
    dinv = lax.rsqrt(deg[...] + 1.0)
    o[...] = jnp.concatenate([dinv * xw[:, 0:H], xw[:, H:]], axis=1)


def _tc_proj1p(x, deg, w):
    return pl.pallas_call(
        _tc_proj1p_body,
        grid=(NP // _RP,),
        in_specs=[pl.BlockSpec((_RP, DI), lambda i: (i, 0)),
                  pl.BlockSpec((_RP, 1), lambda i: (i, 0)),
                  _full((DI, 4 * H))],
        out_specs=pl.BlockSpec((_RP, 4 * H), lambda i: (i, 0)),
        out_shape=jax.ShapeDtypeStruct((NP, 4 * H), jnp.float32),
    )(x, deg, w)


def _tc_proj1a_body(x, w, o):
    o[...] = jnp.dot(x[...], w[...], preferred_element_type=jnp.float32)


def _tc_proj1a(x, w):
    return pl.pallas_call(
        _tc_proj1a_body,
        grid=(NA // _RA,),
        in_specs=[pl.BlockSpec((_RA, DI), lambda i: (i, 0)),
                  _full((DI, 4 * H))],
        out_specs=pl.BlockSpec((_RA, 4 * H), lambda i: (i, 0)),
        out_shape=jax.ShapeDtypeStruct((NA, 4 * H), jnp.float32),
    )(x, w)


def _tc_comb2p_body(accp, xw1, deg, cnt, b, w, o):
    dv = lax.rsqrt(deg[...] + 1.0)
    ic = 1.0 / jnp.maximum(cnt[...], 1.0)
    x1 = xw1[...]
    ac = accp[...]
    p1 = jax.nn.relu(dv * (ac[:, 0:H] + x1[:, 0:H]) + ic * ac[:, H:2 * H]
                     + x1[:, 2 * H:3 * H] + b[...])
    xw = jnp.dot(p1, w[...], preferred_element_type=jnp.float32)
    o[...] = jnp.concatenate([dv * xw[:, 0:H], xw[:, H:]], axis=1)


def _tc_comb2p(accp, xw1, deg, cnt, b, w):
    col = pl.BlockSpec((_RP, 1), lambda i: (i, 0))
    wide = pl.BlockSpec((_RP, 4 * H), lambda i: (i, 0))
    return pl.pallas_call(
        _tc_comb2p_body,
        grid=(NP // _RP,),
        in_specs=[wide, wide, col, col, _full((1, H)), _full((H, 4 * H))],
        out_specs=wide,
        out_shape=jax.ShapeDtypeStruct((NP, 4 * H), jnp.float32),
    )(accp, xw1, deg, cnt, b, w)


def _tc_comb2a_body(acca, xw1, cnt, b, w, o):
    ic = 1.0 / jnp.maximum(cnt[...], 1.0)
    a1 = jax.nn.relu(ic * acca[...][:, 0:H] + xw1[...][:, H:2 * H] + b[...])
    o[...] = jnp.dot(a1, w[...], preferred_element_type=jnp.float32)


def _tc_comb2a(acca, xw1, cnt, b, w):
    col = pl.BlockSpec((_RA, 1), lambda i: (i, 0))
    wide = pl.BlockSpec((_RA, 4 * H), lambda i: (i, 0))
    return pl.pallas_call(
        _tc_comb2a_body,
        grid=(NA // _RA,),
        in_specs=[wide, wide, col, _full((1, H)), _full((H, 4 * H))],
        out_specs=wide,
        out_shape=jax.ShapeDtypeStruct((NA, 4 * H), jnp.float32),
    )(acca, xw1, cnt, b, w)


def _tc_outp_body(accp, xw2, deg, cnt, b, o):
    dv = lax.rsqrt(deg[...] + 1.0)
    ic = 1.0 / jnp.maximum(cnt[...], 1.0)
    x2 = xw2[...]
    ac = accp[...]
    o[...] = (dv * (ac[:, 0:H] + x2[:, 0:H]) + ic * ac[:, H:2 * H]
              + x2[:, 2 * H:3 * H] + b[...])


def _tc_outp(accp, xw2, deg, cnt, b):
    mat = pl.BlockSpec((_RP, H), lambda i: (i, 0))
    col = pl.BlockSpec((_RP, 1), lambda i: (i, 0))
    wide = pl.BlockSpec((_RP, 4 * H), lambda i: (i, 0))
    return pl.pallas_call(
        _tc_outp_body,
        grid=(NP // _RP,),
        in_specs=[wide, wide, col, col, _full((1, H))],
        out_specs=mat,
        out_shape=jax.ShapeDtypeStruct((NP, H), jnp.float32),
    )(accp, xw2, deg, cnt, b)


def _tc_outa_body(acca, xw2, cnt, b, o):
    ic = 1.0 / jnp.maximum(cnt[...], 1.0)
    o[...] = ic * acca[...][:, 0:H] + xw2[...][:, H:2 * H] + b[...]


def _tc_outa(acca, xw2, cnt, b):
    mat = pl.BlockSpec((_RA, H), lambda i: (i, 0))
    col = pl.BlockSpec((_RA, 1), lambda i: (i, 0))
    wide = pl.BlockSpec((_RA, 4 * H), lambda i: (i, 0))
    return pl.pallas_call(
        _tc_outa_body,
        grid=(NA // _RA,),
        in_specs=[wide, wide, col, _full((1, H))],
        out_specs=mat,
        out_shape=jax.ShapeDtypeStruct((NA, H), jnp.float32),
    )(acca, xw2, cnt, b)


def _edge_blocks(src, dst, n_pad, src_mul, src_off, fill_dst):
    n = src.shape[0]
    s = jnp.concatenate([src_mul * src + src_off,
                         jnp.zeros((n_pad - n,), jnp.int32)])
    d = jnp.concatenate([dst, jnp.full((n_pad - n,), fill_dst, jnp.int32)])
    return jnp.stack([s.reshape(-1, BLK), d.reshape(-1, BLK)], axis=1)


def kernel(x_paper, x_author, params, edge_index_pp, edge_index_ap, edge_index_pa):
    P = params
    e_pp = _edge_blocks(edge_index_pp[0], edge_index_pp[1], EPP_PAD, 4, 0, NP)
    e_ap = _edge_blocks(edge_index_ap[0], edge_index_ap[1], EAP_PAD, 4, 0, NP)
    e_pa = _edge_blocks(edge_index_pa[0], edge_index_pa[1], EAP_PAD, 4, 1, NA)

    deg_p, cnt_ap_p, cnt_pa_p = _sc_hist(e_pp, e_ap, e_pa)
    deg = deg_p[:NP, None]
    cnt_ap = cnt_ap_p[:NP, None]
    cnt_pa = cnt_pa_p[:NA, None]

    z128 = jnp.zeros((DI, H), jnp.float32)
    z32 = jnp.zeros((H, H), jnp.float32)
    w1p = jnp.concatenate([P['gcn1_W'], P['pa1_Wl'], P['ap1_Wr'], z128], axis=1)
    w1a = jnp.concatenate([P['ap1_Wl'], P['pa1_Wr'], z128, z128], axis=1)
    w2p = jnp.concatenate([P['gcn2_W'], P['pa2_Wl'], P['ap2_Wr'], z32], axis=1)
    w2a = jnp.concatenate([P['ap2_Wl'], P['pa2_Wr'], z32, z32], axis=1)

    xw1p = _tc_proj1p(x_paper, deg, w1p)
    xw1a = _tc_proj1a(x_author, w1a)

    accp1, acca1 = _sc_scatter(
        xw1p.reshape(4 * NP, H), xw1a.reshape(4 * NA, H), e_pp, e_ap, e_pa)

    b1p = (P['gcn1_b'] + P['ap1_bl'])[None, :]
    b1a = P['pa1_bl'][None, :]
    xw2p = _tc_comb2p(accp1, xw1p, deg, cnt_ap, b1p, w2p)
    xw2a = _tc_comb2a(acca1, xw1a, cnt_pa, b1a, w2a)

    acc_p2, acc_a2 = _sc_scatter(
        xw2p.reshape(4 * NP, H), xw2a.reshape(4 * NA, H), e_pp, e_ap, e_pa)

    b2p = (P['gcn2_b'] + P['ap2_bl'])[None, :]
    b2a = P['pa2_bl'][None, :]
    p2 = _tc_outp(acc_p2, xw2p, deg, cnt_ap, b2p)
    a2 = _tc_outa(acc_a2, xw2a, cnt_pa, b2a)
    return (p2, a2)


# speedup vs baseline: 1.0566x; 1.0566x over previous
import functools

import jax
import jax.numpy as jnp
from jax import lax
from jax.experimental import pallas as pl
from jax.experimental.pallas import tpu as pltpu
from jax.experimental.pallas import tpu_sc as plsc

NP = 50000
NA = 10000
DI = 128
H = 32

NC, NS = 2, 16
BLK = 128

PP_B = 196
AP_B = 52
EPP_PAD = PP_B * BLK * NS
EAP_PAD = AP_B * BLK * NS

NP_H = 50176
NA_H = 10240
PT = 3136
AT = 640

_mesh = plsc.VectorSubcoreMesh(core_axis_name="c", subcore_axis_name="s")
_sc_params = pltpu.CompilerParams(use_tc_tiling_on_sc=False)


def _zero_fill_2d(buf, rows):
    z = jnp.zeros((16,), jnp.float32)

    def st(i, _):
        buf[i, pl.ds(0, 16)] = z
        buf[i, pl.ds(16, 16)] = z
        return 0

    lax.fori_loop(0, rows, st, 0)


def _edge_loop(eidx, tab, acc, b0, nb, ib, rows0, rows1, si, sg0, sg1,
               ss0, ss1):
    ib0, ib1 = ib[0], ib[1]
    si0, si1 = si[0], si[1]
    last = b0 + nb - 1
    pltpu.sync_copy(eidx.at[b0], ib0)
    pltpu.async_copy(tab.at[ib0.at[0]], rows0, sg0)
    pltpu.async_copy(eidx.at[b0 + 1], ib1, si1)

    def pair(i, _):
        b = b0 + 2 * i
        pltpu.make_async_copy(eidx.at[b0], ib1, si1).wait()
        pltpu.async_copy(tab.at[ib1.at[0]], rows1, sg1)
        pltpu.make_async_copy(tab.at[ib0.at[0]], rows0, sg0).wait()
        pltpu.sync_copy(rows0, acc.at[ib0.at[1]], add=True)
        pltpu.async_copy(eidx.at[jnp.minimum(b + 2, last)], ib0, si0)
        pltpu.make_async_copy(eidx.at[b0], ib0, si0).wait()
        pltpu.async_copy(tab.at[ib0.at[0]], rows0, sg0)
        pltpu.make_async_copy(tab.at[ib1.at[0]], rows1, sg1).wait()
        pltpu.sync_copy(rows1, acc.at[ib1.at[1]], add=True)
        pltpu.async_copy(eidx.at[jnp.minimum(b + 3, last)], ib1, si1)
        return 0

    lax.fori_loop(0, nb // 2, pair, 0)
    pltpu.make_async_copy(tab.at[ib0.at[0]], rows0, sg0).wait()
    pltpu.make_async_copy(eidx.at[b0], ib1, si1).wait()


def _hist_loop(eidx, cnt, b0, nb, ib, ones_v, si, ss0, ss1):
    last = b0 + nb - 1

    def iw(k):
        return pltpu.make_async_copy(eidx.at[b0], ib[k], si[k]).wait()

    def scat(k, ss):
        pltpu.async_copy(ones_v, cnt.at[ib[k].at[1]], ss, add=True)

    def sw(k, ss):
        pltpu.make_async_copy(ones_v, cnt.at[ib[k].at[1]], ss).wait()

    for k in range(4):
        pltpu.async_copy(eidx.at[b0 + k], ib[k], si[k])

    def quad(i, _):
        b = b0 + 4 * i
        iw(0)
        scat(0, ss0)
        iw(1)
        scat(1, ss1)
        sw(0, ss0)
        pltpu.async_copy(eidx.at[jnp.minimum(b + 4, last)], ib[0], si[0])
        iw(2)
        scat(2, ss0)
        sw(1, ss1)
        pltpu.async_copy(eidx.at[jnp.minimum(b + 5, last)], ib[1], si[1])
        iw(3)
        scat(3, ss1)
        sw(2, ss0)
        pltpu.async_copy(eidx.at[jnp.minimum(b + 6, last)], ib[2], si[2])
        sw(3, ss1)
        pltpu.async_copy(eidx.at[jnp.minimum(b + 7, last)], ib[3], si[3])
        return 0

    lax.fori_loop(0, nb // 4, quad, 0)
    for k in range(4):
        iw(k)


@functools.partial(
    pl.kernel,
    out_type=(
        jax.ShapeDtypeStruct((NP_H,), jnp.float32),
        jax.ShapeDtypeStruct((NP_H,), jnp.float32),
        jax.ShapeDtypeStruct((NA_H,), jnp.float32),
    ),
    mesh=_mesh,
    compiler_params=_sc_params,
    scratch_types=[
        pltpu.VMEM_SHARED((NP_H,), jnp.float32),
        pltpu.VMEM_SHARED((NA_H,), jnp.float32),
        pltpu.VMEM((2, BLK), jnp.int32),
        pltpu.VMEM((2, BLK), jnp.int32),
        pltpu.VMEM((2, BLK), jnp.int32),
        pltpu.VMEM((2, BLK), jnp.int32),
        pltpu.VMEM((BLK,), jnp.float32),
        pltpu.VMEM((PT,), jnp.float32),
        pltpu.SemaphoreType.DMA,
        pltpu.SemaphoreType.DMA,
        pltpu.SemaphoreType.DMA,
        pltpu.SemaphoreType.DMA,
        pltpu.SemaphoreType.DMA,
        pltpu.SemaphoreType.DMA,
    ],
)
def _sc_hist(e_pp, e_ap, e_pa, out_pp, out_ap, out_pa,
             cntA, cntB, ib0, ib1, ib2, ib3, ones_v, zbuf,
             si0, si1, si2, si3, ss0, ss1):
    ib = [ib0, ib1, ib2, ib3]
    si = [si0, si1, si2, si3]
    c = lax.axis_index("c")
    s = lax.axis_index("s")
    one = jnp.full((16,), 1.0, jnp.float32)
    z = jnp.zeros((16,), jnp.float32)
    for i in range(BLK // 16):
        ones_v[pl.ds(i * 16, 16)] = one

    def zf(i, _):
        zbuf[pl.ds(i * 16, 16)] = z
        return 0

    lax.fori_loop(0, PT // 16, zf, 0)
    pltpu.sync_copy(zbuf, cntA.at[pl.ds(s * PT, PT)])
    pltpu.sync_copy(zbuf.at[pl.ds(0, AT)], cntB.at[pl.ds(s * AT, AT)])
    plsc.subcore_barrier()

    @pl.when(c == 0)
    def _():
        _hist_loop(e_pp, cntA, s * PP_B, PP_B, ib, ones_v, si, ss0, ss1)

    @pl.when(c == 1)
    def _():
        _hist_loop(e_ap, cntA, s * AP_B, AP_B, ib, ones_v, si, ss0, ss1)
        _hist_loop(e_pa, cntB, s * AP_B, AP_B, ib, ones_v, si, ss0, ss1)

    plsc.subcore_barrier()

    @pl.when(c == 0)
    def _():
        pltpu.sync_copy(cntA.at[pl.ds(s * PT, PT)], zbuf)
        pltpu.sync_copy(zbuf, out_pp.at[pl.ds(s * PT, PT)])

    @pl.when(c == 1)
    def _():
        pltpu.sync_copy(cntA.at[pl.ds(s * PT, PT)], zbuf)
        pltpu.sync_copy(zbuf, out_ap.at[pl.ds(s * PT, PT)])
        pltpu.sync_copy(cntB.at[pl.ds(s * AT, AT)], zbuf.at[pl.ds(0, AT)])
        pltpu.sync_copy(zbuf.at[pl.ds(0, AT)], out_pa.at[pl.ds(s * AT, AT)])


@functools.partial(
    pl.kernel,
    out_type=(
        jax.ShapeDtypeStruct((NP_H, 4 * H), jnp.float32),
        jax.ShapeDtypeStruct((NA_H, 4 * H), jnp.float32),
    ),
    mesh=_mesh,
    compiler_params=_sc_params,
    scratch_types=[
        pltpu.VMEM_SHARED((NP_H, H), jnp.float32),
        pltpu.VMEM_SHARED((NA_H, H), jnp.float32),
        pltpu.VMEM((2, BLK), jnp.int32),
        pltpu.VMEM((2, BLK), jnp.int32),
        pltpu.VMEM((2, BLK), jnp.int32),
        pltpu.VMEM((2, BLK), jnp.int32),
        pltpu.VMEM((BLK, H), jnp.float32),
        pltpu.VMEM((BLK, H), jnp.float32),
        pltpu.SemaphoreType.DMA,
        pltpu.SemaphoreType.DMA,
        pltpu.SemaphoreType.DMA,
        pltpu.SemaphoreType.DMA,
        pltpu.SemaphoreType.DMA,
        pltpu.SemaphoreType.DMA,
        pltpu.SemaphoreType.DMA,
        pltpu.SemaphoreType.DMA,
    ],
)
def _sc_scatter(tab_p, tab_a, e_pp, e_ap, e_pa,
                out_p, out_a,
                accA, accB, ib0, ib1, ib2, ib3, rows0, rows1,
                si0, si1, si2, si3, sg0, sg1, ss0, ss1):
    ib = [ib0, ib1, ib2, ib3]
    si = [si0, si1, si2, si3]
    c = lax.axis_index("c")
    s = lax.axis_index("s")
    _zero_fill_2d(rows0, BLK)
    zcp = ([(accA, s * PT + k * BLK, BLK) for k in range(PT // BLK)]
           + [(accA, s * PT + (PT // BLK) * BLK, PT % BLK)]
           + [(accB, s * AT + k * BLK, BLK) for k in range(AT // BLK)])
    for acc, off, sz in zcp:
        pltpu.async_copy(rows0.at[pl.ds(0, sz)], acc.at[pl.ds(off, sz)], sg0)
    for acc, off, sz in zcp:
        pltpu.make_async_copy(rows0.at[pl.ds(0, sz)],
                              acc.at[pl.ds(off, sz)], sg0).wait()
    plsc.subcore_barrier()

    @pl.when(c == 0)
    def _():
        _edge_loop(e_pp, tab_p, accA, s * PP_B, PP_B,
                   ib, rows0, rows1, si, sg0, sg1, ss0, ss1)

    @pl.when(c == 1)
    def _():
        _edge_loop(e_ap, tab_a, accA, s * AP_B, AP_B,
                   ib, rows0, rows1, si, sg0, sg1, ss0, ss1)
        _edge_loop(e_pa, tab_p, accB, s * AP_B, AP_B,
                   ib, rows0, rows1, si, sg0, sg1, ss0, ss1)

    plsc.subcore_barrier()

    def _drain(jobs):
        bufs, rsem, wsem = (rows0, rows1), (sg0, sg1), (ss0, ss1)

        def rd(j, job):
            acc, out, col, off, sz = job
            pltpu.async_copy(acc.at[pl.ds(off, sz)],
                             bufs[j % 2].at[pl.ds(0, sz)], rsem[j % 2])

        def wr(j, job, wait_only):
            acc, out, col, off, sz = job
            d = (pltpu.make_async_copy if wait_only else pltpu.async_copy)(
                bufs[j % 2].at[pl.ds(0, sz)],
                out.at[pl.ds(off, sz), pl.ds(col, H)], wsem[j % 2])
            if wait_only:
                d.wait()

        for j, job in enumerate(jobs):
            if j >= 2:
                wr(j - 2, jobs[j - 2], True)
            rd(j, job)
            pltpu.make_async_copy(
                job[0].at[pl.ds(job[3], job[4])],
                bufs[j % 2].at[pl.ds(0, job[4])], rsem[j % 2]).wait()
            wr(j, job, False)
        for j in range(max(0, len(jobs) - 2), len(jobs)):
            wr(j, jobs[j], True)

    def _jobs(acc, out, col, base, n):
        js = [(acc, out, col, base + k * BLK, BLK) for k in range(n // BLK)]
        if n % BLK:
            js.append((acc, out, col, base + (n // BLK) * BLK, n % BLK))
        return js

    @pl.when(c == 0)
    def _():
        _drain(_jobs(accA, out_p, 0, s * PT, PT))

    @pl.when(c == 1)
    def _():
        _drain(_jobs(accA, out_p, H, s * PT, PT)
               + _jobs(accB, out_a, 0, s * AT, AT))


_RP = 5000
_RA = 5000


def _full(shape):
    return pl.BlockSpec(shape, lambda i: (0, 0))


def _tc_proj1p_body(x, deg, w, o):
    xw = jnp.dot(x[...], w[...], preferred_element_type=jnp.float32)
    dinv = lax.rsqrt(deg[...] + 1.0)
    o[...] = jnp.concatenate([dinv * xw[:, 0:H], xw[:, H:]], axis=1)


def _tc_proj1p(x, deg, w):
    return pl.pallas_call(
        _tc_proj1p_body,
        grid=(NP // _RP,),
        in_specs=[pl.BlockSpec((_RP, DI), lambda i: (i, 0)),
                  pl.BlockSpec((_RP, 1), lambda i: (i, 0)),
                  _full((DI, 4 * H))],
        out_specs=pl.BlockSpec((_RP, 4 * H), lambda i: (i, 0)),
        out_shape=jax.ShapeDtypeStruct((NP, 4 * H), jnp.float32),
    )(x, deg, w)


def _tc_proj1a_body(x, w, o):
    o[...] = jnp.dot(x[...], w[...], preferred_element_type=jnp.float32)


def _tc_proj1a(x, w):
    return pl.pallas_call(
        _tc_proj1a_body,
        grid=(NA // _RA,),
        in_specs=[pl.BlockSpec((_RA, DI), lambda i: (i, 0)),
                  _full((DI, 4 * H))],
        out_specs=pl.BlockSpec((_RA, 4 * H), lambda i: (i, 0)),
        out_shape=jax.ShapeDtypeStruct((NA, 4 * H), jnp.float32),
    )(x, w)


def _tc_comb2p_body(accp, xw1, deg, cnt, b, w, o):
    dv = lax.rsqrt(deg[...] + 1.0)
    ic = 1.0 / jnp.maximum(cnt[...], 1.0)
    x1 = xw1[...]
    ac = accp[...]
    p1 = jax.nn.relu(dv * (ac[:, 0:H] + x1[:, 0:H]) + ic * ac[:, H:2 * H]
                     + x1[:, 2 * H:3 * H] + b[...])
    xw = jnp.dot(p1, w[...], preferred_element_type=jnp.float32)
    o[...] = jnp.concatenate([dv * xw[:, 0:H], xw[:, H:]], axis=1)


def _tc_comb2p(accp, xw1, deg, cnt, b, w):
    col = pl.BlockSpec((_RP, 1), lambda i: (i, 0))
    wide = pl.BlockSpec((_RP, 4 * H), lambda i: (i, 0))
    return pl.pallas_call(
        _tc_comb2p_body,
        grid=(NP // _RP,),
        in_specs=[wide, wide, col, col, _full((1, H)), _full((H, 4 * H))],
        out_specs=wide,
        out_shape=jax.ShapeDtypeStruct((NP, 4 * H), jnp.float32),
    )(accp, xw1, deg, cnt, b, w)


def _tc_comb2a_body(acca, xw1, cnt, b, w, o):
    ic = 1.0 / jnp.maximum(cnt[...], 1.0)
    a1 = jax.nn.relu(ic * acca[...][:, 0:H] + xw1[...][:, H:2 * H] + b[...])
    o[...] = jnp.dot(a1, w[...], preferred_element_type=jnp.float32)


def _tc_comb2a(acca, xw1, cnt, b, w):
    col = pl.BlockSpec((_RA, 1), lambda i: (i, 0))
    wide = pl.BlockSpec((_RA, 4 * H), lambda i: (i, 0))
    return pl.pallas_call(
        _tc_comb2a_body,
        grid=(NA // _RA,),
        in_specs=[wide, wide, col, _full((1, H)), _full((H, 4 * H))],
        out_specs=wide,
        out_shape=jax.ShapeDtypeStruct((NA, 4 * H), jnp.float32),
    )(acca, xw1, cnt, b, w)


def _tc_outp_body(accp, xw2, deg, cnt, b, o):
    dv = lax.rsqrt(deg[...] + 1.0)
    ic = 1.0 / jnp.maximum(cnt[...], 1.0)
    x2 = xw2[...]
    ac = accp[...]
    o[...] = (dv * (ac[:, 0:H] + x2[:, 0:H]) + ic * ac[:, H:2 * H]
              + x2[:, 2 * H:3 * H] + b[...])


def _tc_outp(accp, xw2, deg, cnt, b):
    mat = pl.BlockSpec((_RP, H), lambda i: (i, 0))
    col = pl.BlockSpec((_RP, 1), lambda i: (i, 0))
    wide = pl.BlockSpec((_RP, 4 * H), lambda i: (i, 0))
    return pl.pallas_call(
        _tc_outp_body,
        grid=(NP // _RP,),
        in_specs=[wide, wide, col, col, _full((1, H))],
        out_specs=mat,
        out_shape=jax.ShapeDtypeStruct((NP, H), jnp.float32),
    )(accp, xw2, deg, cnt, b)


def _tc_outa_body(acca, xw2, cnt, b, o):
    ic = 1.0 / jnp.maximum(cnt[...], 1.0)
    o[...] = ic * acca[...][:, 0:H] + xw2[...][:, H:2 * H] + b[...]


def _tc_outa(acca, xw2, cnt, b):
    mat = pl.BlockSpec((_RA, H), lambda i: (i, 0))
    col = pl.BlockSpec((_RA, 1), lambda i: (i, 0))
    wide = pl.BlockSpec((_RA, 4 * H), lambda i: (i, 0))
    return pl.pallas_call(
        _tc_outa_body,
        grid=(NA // _RA,),
        in_specs=[wide, wide, col, _full((1, H))],
        out_specs=mat,
        out_shape=jax.ShapeDtypeStruct((NA, H), jnp.float32),
    )(acca, xw2, cnt, b)


def _edge_blocks(src, dst, n_pad, src_mul, src_off, fill_dst):
    n = src.shape[0]
    s = jnp.concatenate([src_mul * src + src_off,
                         jnp.zeros((n_pad - n,), jnp.int32)])
    d = jnp.concatenate([dst, jnp.full((n_pad - n,), fill_dst, jnp.int32)])
    return jnp.stack([s.reshape(-1, BLK), d.reshape(-1, BLK)], axis=1)


def kernel(x_paper, x_author, params, edge_index_pp, edge_index_ap, edge_index_pa):
    P = params
    e_pp = _edge_blocks(edge_index_pp[0], edge_index_pp[1], EPP_PAD, 4, 0, NP)
    e_ap = _edge_blocks(edge_index_ap[0], edge_index_ap[1], EAP_PAD, 4, 0, NP)
    e_pa = _edge_blocks(edge_index_pa[0], edge_index_pa[1], EAP_PAD, 4, 1, NA)

    deg_p, cnt_ap_p, cnt_pa_p = _sc_hist(e_pp, e_ap, e_pa)
    deg = deg_p[:NP, None]
    cnt_ap = cnt_ap_p[:NP, None]
    cnt_pa = cnt_pa_p[:NA, None]

    z128 = jnp.zeros((DI, H), jnp.float32)
    z32 = jnp.zeros((H, H), jnp.float32)
    w1p = jnp.concatenate([P['gcn1_W'], P['pa1_Wl'], P['ap1_Wr'], z128], axis=1)
    w1a = jnp.concatenate([P['ap1_Wl'], P['pa1_Wr'], z128, z128], axis=1)
    w2p = jnp.concatenate([P['gcn2_W'], P['pa2_Wl'], P['ap2_Wr'], z32], axis=1)
    w2a = jnp.concatenate([P['ap2_Wl'], P['pa2_Wr'], z32, z32], axis=1)

    xw1p = _tc_proj1p(x_paper, deg, w1p)
    xw1a = _tc_proj1a(x_author, w1a)

    accp1, acca1 = _sc_scatter(
        xw1p.reshape(4 * NP, H), xw1a.reshape(4 * NA, H), e_pp, e_ap, e_pa)

    b1p = (P['gcn1_b'] + P['ap1_bl'])[None, :]
    b1a = P['pa1_bl'][None, :]
    xw2p = _tc_comb2p(accp1, xw1p, deg, cnt_ap, b1p, w2p)
    xw2a = _tc_comb2a(acca1, xw1a, cnt_pa, b1a, w2a)

    acc_p2, acc_a2 = _sc_scatter(
        xw2p.reshape(4 * NP, H), xw2a.reshape(4 * NA, H), e_pp, e_ap, e_pa)

    b2p = (P['gcn2_b'] + P['ap2_bl'])[None, :]
    b2a = P['pa2_bl'][None, :]
    p2 = _tc_outp(acc_p2, xw2p, deg, cnt_ap, b2p)
    a2 = _tc_outa(acc_a2, xw2a, cnt_pa, b2a)
    return (p2, a2)
